# baseline (speedup 1.0000x reference)
import jax
import jax.numpy as jnp
from jax import lax
from jax.experimental import pallas as pl
from jax.experimental.pallas import tpu as pltpu
from jax.experimental.pallas import tpu_sc as plsc

NC = 2
NS = 16
NW = NC * NS
N_NODES = 10000
D = 128

CHUNK = 64
N_CHUNK_ROWS = 5000
BASE_CHUNKS = 152
GROUP = 8
RING = 5
AHEAD = 4
N_EXTRA = N_CHUNK_ROWS - NW * BASE_CHUNKS
EXTRA_TILES = N_EXTRA // GROUP


def _sc_body(x_hbm, e_hbm, out_hbm, acc, estage, didx, *rest):
    rows = rest[:RING]
    gsems = rest[RING:2 * RING]
    ssems = rest[2 * RING:3 * RING]

    c = lax.axis_index("c")
    s = lax.axis_index("s")
    wid = s * NC + c

    @pl.when(s < 15)
    def _():
        pltpu.sync_copy(x_hbm.at[pl.ds(s * 632, 632)],
                        acc.at[pl.ds(s * 632, 632)])

    @pl.when(s == 15)
    def _():
        pltpu.sync_copy(x_hbm.at[pl.ds(9480, 520)],
                        acc.at[pl.ds(9480, 520)])

    plsc.subcore_barrier()

    def run_chunks(chunk_base, n_chunks, gcps, scps):
        def fire_gather(f):
            b = f % RING
            if f >= RING:
                scps[f - RING].wait()
            if f % GROUP == 0:
                pltpu.sync_copy(e_hbm.at[:, pl.ds(chunk_base + f, GROUP)],
                                estage)
            for v in range(CHUNK // 16):
                didx[b, pl.ds(v * 16, 16)] = estage[1, f % GROUP,
                                                    pl.ds(v * 16, 16)]
            gcps[f] = pltpu.async_copy(
                x_hbm.at[estage.at[0, f % GROUP]], rows[b], gsems[b])

        def fire_scatter(i):
            b = i % RING
            gcps[i].wait()
            scps[i] = pltpu.async_copy(
                rows[b], acc.at[didx.at[b]], ssems[b], add=True)

        for i in range(-AHEAD, n_chunks):
            f = i + AHEAD
            if f < n_chunks:
                fire_gather(f)
            if i >= 0:
                fire_scatter(i)
        for i in range(max(0, n_chunks - RING), n_chunks):
            scps[i].wait()

    gcps = [None] * BASE_CHUNKS
    scps = [None] * BASE_CHUNKS
    run_chunks(wid * BASE_CHUNKS, BASE_CHUNKS, gcps, scps)

    @pl.when(wid < EXTRA_TILES)
    def _():
        gcps2 = [None] * GROUP
        scps2 = [None] * GROUP
        run_chunks(NW * BASE_CHUNKS + wid * GROUP, GROUP, gcps2, scps2)

    plsc.subcore_barrier()

    @pl.when(s < 15)
    def _():
        pltpu.sync_copy(acc.at[pl.ds(s * 632, 632)],
                        out_hbm.at[c, pl.ds(s * 632, 632)])

    @pl.when(s == 15)
    def _():
        pltpu.sync_copy(acc.at[pl.ds(9480, 520)],
                        out_hbm.at[c, pl.ds(9480, 520)])


def _combine_body(x_ref, p0_ref, p1_ref, o_ref):
    o_ref[...] = p0_ref[0] + p1_ref[0] - x_ref[...]


@jax.jit
def kernel(x, edge_index):
    e3 = edge_index.reshape(NC, N_CHUNK_ROWS, CHUNK)

    mesh = plsc.VectorSubcoreMesh(
        core_axis_name="c", subcore_axis_name="s",
        num_cores=NC, num_subcores=NS)

    partials = pl.kernel(
        _sc_body,
        out_type=jax.ShapeDtypeStruct((NC, N_NODES, D), jnp.float32),
        mesh=mesh,
        scratch_types=(
            [pltpu.VMEM_SHARED((N_NODES, D), jnp.float32),
             pltpu.VMEM((2, GROUP, CHUNK), jnp.int32),
             pltpu.VMEM((RING, CHUNK), jnp.int32)]
            + [pltpu.VMEM((CHUNK, D), jnp.float32)] * RING
            + [pltpu.SemaphoreType.DMA] * (2 * RING)
        ),
    )(x, e3)

    blk = 2000
    out = pl.pallas_call(
        _combine_body,
        grid=(N_NODES // blk,),
        in_specs=[
            pl.BlockSpec((blk, D), lambda i: (i, 0)),
            pl.BlockSpec((1, blk, D), lambda i: (0, i, 0)),
            pl.BlockSpec((1, blk, D), lambda i: (1, i, 0)),
        ],
        out_specs=pl.BlockSpec((blk, D), lambda i: (i, 0)),
        out_shape=jax.ShapeDtypeStruct((N_NODES, D), jnp.float32),
    )(x, partials, partials)
    return out


# speedup vs baseline: 1.0213x; 1.0213x over previous
import jax
import jax.numpy as jnp
from jax import lax
from jax.experimental import pallas as pl
from jax.experimental.pallas import tpu as pltpu
from jax.experimental.pallas import tpu_sc as plsc

NC = 2
NS = 16
NW = NC * NS
N_NODES = 10000
D = 128

CHUNK = 128
N_CHUNK_ROWS = 2500
BASE_CHUNKS = 76
GROUP = 4
RING = 3
AHEAD = 2
N_EXTRA = N_CHUNK_ROWS - NW * BASE_CHUNKS
EXTRA_TILES = N_EXTRA // GROUP


def _sc_body(x_hbm, e_hbm, out_hbm, acc, estage, didx, rows0, rows1, rows2,
             gs0, gs1, gs2, ss0, ss1, ss2):
    c = lax.axis_index("c")
    s = lax.axis_index("s")
    wid = s * NC + c

    @pl.when(s < 15)
    def _():
        pltpu.sync_copy(x_hbm.at[pl.ds(s * 632, 632)],
                        acc.at[pl.ds(s * 632, 632)])

    @pl.when(s == 15)
    def _():
        pltpu.sync_copy(x_hbm.at[pl.ds(9480, 520)],
                        acc.at[pl.ds(9480, 520)])

    plsc.subcore_barrier()

    rows = (rows0, rows1, rows2)
    gsems = (gs0, gs1, gs2)
    ssems = (ss0, ss1, ss2)

    def run_chunks(chunk_base, n_chunks, gcps, scps):
        def fire_gather(f):
            b = f % RING
            if f >= RING:
                scps[f - RING].wait()
            if f % GROUP == 0:
                pltpu.sync_copy(e_hbm.at[:, pl.ds(chunk_base + f, GROUP)],
                                estage)
            for v in range(CHUNK // 16):
                didx[b, pl.ds(v * 16, 16)] = estage[1, f % GROUP,
                                                    pl.ds(v * 16, 16)]
            gcps[f] = pltpu.async_copy(
                x_hbm.at[estage.at[0, f % GROUP]], rows[b], gsems[b])

        def fire_scatter(i):
            b = i % RING
            gcps[i].wait()
            scps[i] = pltpu.async_copy(
                rows[b], acc.at[didx.at[b]], ssems[b], add=True)

        for i in range(-AHEAD, n_chunks):
            f = i + AHEAD
            if f < n_chunks:
                fire_gather(f)
            if i >= 0:
                fire_scatter(i)
        for i in range(max(0, n_chunks - RING), n_chunks):
            scps[i].wait()

    gcps = [None] * BASE_CHUNKS
    scps = [None] * BASE_CHUNKS
    run_chunks(wid * BASE_CHUNKS, BASE_CHUNKS, gcps, scps)

    @pl.when(wid < EXTRA_TILES)
    def _():
        gcps2 = [None] * GROUP
        scps2 = [None] * GROUP
        run_chunks(NW * BASE_CHUNKS + wid * GROUP, GROUP, gcps2, scps2)

    plsc.subcore_barrier()

    @pl.when(s < 15)
    def _():
        pltpu.sync_copy(acc.at[pl.ds(s * 632, 632)],
                        out_hbm.at[c, pl.ds(s * 632, 632)])

    @pl.when(s == 15)
    def _():
        pltpu.sync_copy(acc.at[pl.ds(9480, 520)],
                        out_hbm.at[c, pl.ds(9480, 520)])


def _combine_body(x_ref, p0_ref, p1_ref, o_ref):
    o_ref[...] = p0_ref[0] + p1_ref[0] - x_ref[...]


@jax.jit
def kernel(x, edge_index):
    e3 = edge_index.reshape(NC, N_CHUNK_ROWS, CHUNK)

    mesh = plsc.VectorSubcoreMesh(
        core_axis_name="c", subcore_axis_name="s",
        num_cores=NC, num_subcores=NS)

    partials = pl.kernel(
        _sc_body,
        out_type=jax.ShapeDtypeStruct((NC, N_NODES, D), jnp.float32),
        mesh=mesh,
        scratch_types=[
            pltpu.VMEM_SHARED((N_NODES, D), jnp.float32),
            pltpu.VMEM((2, GROUP, CHUNK), jnp.int32),
            pltpu.VMEM((RING, CHUNK), jnp.int32),
            pltpu.VMEM((CHUNK, D), jnp.float32),
            pltpu.VMEM((CHUNK, D), jnp.float32),
            pltpu.VMEM((CHUNK, D), jnp.float32),
            pltpu.SemaphoreType.DMA,
            pltpu.SemaphoreType.DMA,
            pltpu.SemaphoreType.DMA,
            pltpu.SemaphoreType.DMA,
            pltpu.SemaphoreType.DMA,
            pltpu.SemaphoreType.DMA,
        ],
    )(x, e3)

    blk = 5000
    out = pl.pallas_call(
        _combine_body,
        grid=(N_NODES // blk,),
        in_specs=[
            pl.BlockSpec((blk, D), lambda i: (i, 0)),
            pl.BlockSpec((1, blk, D), lambda i: (0, i, 0)),
            pl.BlockSpec((1, blk, D), lambda i: (1, i, 0)),
        ],
        out_specs=pl.BlockSpec((blk, D), lambda i: (i, 0)),
        out_shape=jax.ShapeDtypeStruct((N_NODES, D), jnp.float32),
    )(x, partials, partials)
    return out
